# baseline (speedup 1.0000x reference)
import functools

import jax
import jax.numpy as jnp
from jax import lax
from jax.experimental import pallas as pl
from jax.experimental.pallas import tpu as pltpu
from jax.experimental.pallas import tpu_sc as plsc

EPS = 0.1
T_TILE = 2048
V_TILE = 2048

N_TOK = 16384
N_VOCAB = 8192
D = 64

_NC, _NS = 2, 16
_NW = _NC * _NS
_BPW = N_TOK // _NW


NT = N_TOK // T_TILE
NV = N_VOCAB // V_TILE


def _argmax_body(flat_ref, vocab_ref, ids_ref, voc_scr):
    t = pl.program_id(0)

    @pl.when(t == 0)
    def _():
        voc = vocab_ref[...]
        vn = jnp.sqrt(jnp.sum(voc * voc, axis=1, keepdims=True))
        voc_scr[...] = (voc / jnp.maximum(vn, 1e-12)).astype(jnp.bfloat16)

    emb = flat_ref[...]
    en = jnp.sqrt(jnp.sum(emb * emb, axis=1, keepdims=True))
    emb_n = (emb / jnp.maximum(en, 1e-12)).astype(jnp.bfloat16)

    rm = jnp.full((T_TILE, 128), -jnp.inf, jnp.float32)
    ri = jnp.zeros((T_TILE, 128), jnp.float32)
    for v in range(NV):
        sim = jax.lax.dot_general(
            emb_n, voc_scr[v * V_TILE:(v + 1) * V_TILE, :],
            (((1,), (1,)), ((), ())),
            preferred_element_type=jnp.float32,
        )
        for c in range(V_TILE // 128):
            chunk = sim[:, c * 128:(c + 1) * 128]
            gt = chunk > rm
            ri = jnp.where(gt, jnp.float32(v * (V_TILE // 128) + c), ri)
            rm = jnp.where(gt, chunk, rm)

    m = jnp.max(rm, axis=1, keepdims=True)
    lane_f = jax.lax.broadcasted_iota(
        jnp.int32, (1, 128), 1).astype(jnp.float32)
    enc = ri * jnp.float32(128.0) + lane_f
    idx_f = jnp.min(jnp.where(rm == m, enc, jnp.float32(jnp.inf)),
                    axis=1, keepdims=True)
    ids_ref[...] = idx_f.astype(jnp.int32)


def _anchor_ids(flat, vocab):
    n_tok = flat.shape[0]
    ids = pl.pallas_call(
        _argmax_body,
        grid=(n_tok // T_TILE,),
        in_specs=[
            pl.BlockSpec((T_TILE, D), lambda t: (t, 0)),
            pl.BlockSpec((N_VOCAB, D), lambda t: (0, 0)),
        ],
        out_specs=pl.BlockSpec((T_TILE, 1), lambda t: (t, 0)),
        out_shape=jax.ShapeDtypeStruct((n_tok, 1), jnp.int32),
        scratch_shapes=[
            pltpu.VMEM((N_VOCAB, D), jnp.bfloat16),
        ],
    )(flat, vocab)
    return ids[:, 0]


def _make_gather_body(bpw):
    def _gather_body(ids_hbm, vocab_hbm, out_hbm, idx_v, rows_v, sem):
        wid = lax.axis_index("s") * _NC + lax.axis_index("c")
        base = wid * bpw
        pltpu.sync_copy(ids_hbm.at[pl.ds(base, bpw)], idx_v)
        pltpu.async_copy(vocab_hbm.at[idx_v], rows_v, sem).wait()
        pltpu.sync_copy(rows_v, out_hbm.at[pl.ds(base, bpw)])
    return _gather_body


@functools.cache
def _sc_gather_kernel(n_tok):
    bpw = n_tok // _NW
    return pl.kernel(
        _make_gather_body(bpw),
        out_type=jax.ShapeDtypeStruct((n_tok, 128), jnp.float32),
        mesh=plsc.VectorSubcoreMesh(core_axis_name="c", subcore_axis_name="s"),
        scratch_types=[
            pltpu.VMEM((bpw,), jnp.int32),
            pltpu.VMEM((bpw, 128), jnp.float32),
            pltpu.SemaphoreType.DMA,
        ],
    )


def _clip_body(flat_ref, anc_ref, out_ref):
    f = flat_ref[...]
    a = anc_ref[:, :D]
    off = f - a
    on2 = jnp.sum(off * off, axis=1, keepdims=True)
    an2 = jnp.sum(a * a, axis=1, keepdims=True)
    scale = jnp.minimum(EPS * jnp.sqrt(an2) / (jnp.sqrt(on2) + 1e-8), 1.0)
    out_ref[...] = a + off * scale


def _clip(flat, anchors):
    c_tile = 2048
    n_tok = flat.shape[0]
    return pl.pallas_call(
        _clip_body,
        grid=(n_tok // c_tile,),
        in_specs=[
            pl.BlockSpec((c_tile, D), lambda t: (t, 0)),
            pl.BlockSpec((c_tile, 128), lambda t: (t, 0)),
        ],
        out_specs=pl.BlockSpec((c_tile, D), lambda t: (t, 0)),
        out_shape=jax.ShapeDtypeStruct((n_tok, D), jnp.float32),
    )(flat, anchors)


def kernel(embeddings, vocab_embeddings):
    B, S, _ = embeddings.shape
    flat = embeddings.reshape(-1, D)
    vocab_pad = jnp.pad(vocab_embeddings, ((0, 0), (0, 128 - D)))
    half = N_TOK // 2
    flat_a, flat_b = flat[:half], flat[half:]
    ids_a = _anchor_ids(flat_a, vocab_embeddings)
    anchors_a = _sc_gather_kernel(half)(ids_a, vocab_pad)
    ids_b = _anchor_ids(flat_b, vocab_embeddings)
    anchors_b = _sc_gather_kernel(half)(ids_b, vocab_pad)
    res_a = _clip(flat_a, anchors_a)
    res_b = _clip(flat_b, anchors_b)
    res = jnp.concatenate([res_a, res_b], axis=0)
    ids = jnp.concatenate([ids_a, ids_b], axis=0)
    return res.reshape(B, S, D), ids.reshape(B, S)


# speedup vs baseline: 1.1879x; 1.1879x over previous
import functools

import jax
import jax.numpy as jnp
from jax import lax
from jax.experimental import pallas as pl
from jax.experimental.pallas import tpu as pltpu
from jax.experimental.pallas import tpu_sc as plsc

EPS = 0.1
T_TILE = 2048
V_TILE = 2048

N_TOK = 16384
N_VOCAB = 8192
D = 64

_NC, _NS = 2, 16
_NW = _NC * _NS
_BPW = N_TOK // _NW


NT = N_TOK // T_TILE
NV = N_VOCAB // V_TILE


def _argmax_body(flat_ref, vocab_ref, ids_ref, vpad_ref, voc_scr):
    t = pl.program_id(0)

    @pl.when(t == 0)
    def _():
        voc = vocab_ref[...]
        vn = jnp.sqrt(jnp.sum(voc * voc, axis=1, keepdims=True))
        voc_scr[...] = (voc / jnp.maximum(vn, 1e-12)).astype(jnp.bfloat16)
        vpad_ref[:, :D] = voc
        vpad_ref[:, D:] = jnp.zeros((N_VOCAB, 128 - D), jnp.float32)

    emb = flat_ref[...]
    en = jnp.sqrt(jnp.sum(emb * emb, axis=1, keepdims=True))
    emb_n = (emb / jnp.maximum(en, 1e-12)).astype(jnp.bfloat16)

    rm = jnp.full((T_TILE, 128), -jnp.inf, jnp.float32)
    ri = jnp.zeros((T_TILE, 128), jnp.float32)
    for v in range(NV):
        sim = jax.lax.dot_general(
            emb_n, voc_scr[v * V_TILE:(v + 1) * V_TILE, :],
            (((1,), (1,)), ((), ())),
            preferred_element_type=jnp.float32,
        )
        for c in range(V_TILE // 128):
            chunk = sim[:, c * 128:(c + 1) * 128]
            gt = chunk > rm
            ri = jnp.where(gt, jnp.float32(v * (V_TILE // 128) + c), ri)
            rm = jnp.where(gt, chunk, rm)

    m = jnp.max(rm, axis=1, keepdims=True)
    lane_f = jax.lax.broadcasted_iota(
        jnp.int32, (1, 128), 1).astype(jnp.float32)
    enc = ri * jnp.float32(128.0) + lane_f
    idx_f = jnp.min(jnp.where(rm == m, enc, jnp.float32(jnp.inf)),
                    axis=1, keepdims=True)
    ids_ref[...] = idx_f.astype(jnp.int32)


def _anchor_ids(flat, vocab):
    ids, vpad = pl.pallas_call(
        _argmax_body,
        grid=(NT,),
        in_specs=[
            pl.BlockSpec((T_TILE, D), lambda t: (t, 0)),
            pl.BlockSpec((N_VOCAB, D), lambda t: (0, 0)),
        ],
        out_specs=[
            pl.BlockSpec((T_TILE, 1), lambda t: (t, 0)),
            pl.BlockSpec((N_VOCAB, 128), lambda t: (0, 0)),
        ],
        out_shape=[
            jax.ShapeDtypeStruct((N_TOK, 1), jnp.int32),
            jax.ShapeDtypeStruct((N_VOCAB, 128), jnp.float32),
        ],
        scratch_shapes=[
            pltpu.VMEM((N_VOCAB, D), jnp.bfloat16),
        ],
    )(flat, vocab)
    return ids[:, 0], vpad


def _gather_body(ids_hbm, vocab_hbm, out_hbm, idx_v, rows_v, sem):
    wid = lax.axis_index("s") * _NC + lax.axis_index("c")
    base = wid * _BPW
    pltpu.sync_copy(ids_hbm.at[pl.ds(base, _BPW)], idx_v)
    pltpu.async_copy(vocab_hbm.at[idx_v], rows_v, sem).wait()
    pltpu.sync_copy(rows_v, out_hbm.at[pl.ds(base, _BPW)])


@functools.cache
def _sc_gather_kernel():
    return pl.kernel(
        _gather_body,
        out_type=jax.ShapeDtypeStruct((N_TOK, 128), jnp.float32),
        mesh=plsc.VectorSubcoreMesh(core_axis_name="c", subcore_axis_name="s"),
        scratch_types=[
            pltpu.VMEM((_BPW,), jnp.int32),
            pltpu.VMEM((_BPW, 128), jnp.float32),
            pltpu.SemaphoreType.DMA,
        ],
    )


def _clip_body(flat_ref, anc_ref, out_ref):
    f = flat_ref[...]
    a = anc_ref[:, :D]
    off = f - a
    on2 = jnp.sum(off * off, axis=1, keepdims=True)
    an2 = jnp.sum(a * a, axis=1, keepdims=True)
    scale = jnp.minimum(EPS * jnp.sqrt(an2) / (jnp.sqrt(on2) + 1e-8), 1.0)
    out_ref[...] = a + off * scale


def _clip(flat, anchors):
    c_tile = 2048
    return pl.pallas_call(
        _clip_body,
        grid=(N_TOK // c_tile,),
        in_specs=[
            pl.BlockSpec((c_tile, D), lambda t: (t, 0)),
            pl.BlockSpec((c_tile, 128), lambda t: (t, 0)),
        ],
        out_specs=pl.BlockSpec((c_tile, D), lambda t: (t, 0)),
        out_shape=jax.ShapeDtypeStruct((N_TOK, D), jnp.float32),
    )(flat, anchors)


def kernel(embeddings, vocab_embeddings):
    B, S, _ = embeddings.shape
    flat = embeddings.reshape(-1, D)
    ids, vocab_pad = _anchor_ids(flat, vocab_embeddings)
    anchors = _sc_gather_kernel()(ids, vocab_pad)
    res = _clip(flat, anchors)
    return res.reshape(B, S, D), ids.reshape(B, S)
